# Initial kernel scaffold; baseline (speedup 1.0000x reference)
import jax
import jax.numpy as jnp
from jax.experimental import pallas as pl


def kernel(bboxes, gt_bboxes, gt_labels):
    raise NotImplementedError("write your pallas kernel here")


# speedup vs baseline: 2.9509x; 2.9509x over previous
import functools

import jax
import jax.numpy as jnp
from jax import lax
from jax.experimental import pallas as pl
from jax.experimental.pallas import tpu as pltpu
from jax.experimental.pallas import tpu_sc as plsc

POS_IOU_THR = 0.5
NEG_IOU_THR = 0.4

N = 20000
M = 128
LANES = 16
NWORKERS = 32
NPAD = 20480
APW = NPAD // NWORKERS
CPW = APW // LANES

_mesh = plsc.VectorSubcoreMesh(core_axis_name="c", subcore_axis_name="s")


@functools.partial(
    pl.kernel,
    mesh=_mesh,
    compiler_params=pltpu.CompilerParams(needs_layout_passes=False),
    out_type=[
        jax.ShapeDtypeStruct((NPAD,), jnp.int32),
        jax.ShapeDtypeStruct((NPAD * 4,), jnp.float32),
    ],
    scratch_types=[
        pltpu.VMEM((APW,), jnp.float32),
        pltpu.VMEM((APW,), jnp.float32),
        pltpu.VMEM((APW,), jnp.float32),
        pltpu.VMEM((APW,), jnp.float32),
        pltpu.VMEM((M,), jnp.float32),
        pltpu.VMEM((M,), jnp.float32),
        pltpu.VMEM((M,), jnp.float32),
        pltpu.VMEM((M,), jnp.float32),
        pltpu.VMEM((M,), jnp.float32),
        pltpu.VMEM((M,), jnp.int32),
        pltpu.VMEM((APW,), jnp.int32),
        pltpu.VMEM((APW * 4,), jnp.float32),
    ],
)
def _assign(bx1_h, by1_h, bx2_h, by2_h, gx1_h, gy1_h, gx2_h, gy2_h, glab_h,
            olab_h, obox_h,
            vx1, vy1, vx2, vy2, vgx1, vgy1, vgx2, vgy2, vgarea, vglab,
            vlab, vbox):
    wid = lax.axis_index("s") * 2 + lax.axis_index("c")
    base = wid * APW

    pltpu.sync_copy(bx1_h.at[pl.ds(base, APW)], vx1)
    pltpu.sync_copy(by1_h.at[pl.ds(base, APW)], vy1)
    pltpu.sync_copy(bx2_h.at[pl.ds(base, APW)], vx2)
    pltpu.sync_copy(by2_h.at[pl.ds(base, APW)], vy2)
    pltpu.sync_copy(gx1_h, vgx1)
    pltpu.sync_copy(gy1_h, vgy1)
    pltpu.sync_copy(gx2_h, vgx2)
    pltpu.sync_copy(gy2_h, vgy2)
    pltpu.sync_copy(glab_h, vglab)

    zero = jnp.float32(0.0)
    for s in range(M // LANES):
        g1 = vgx1[pl.ds(s * LANES, LANES)]
        g2 = vgy1[pl.ds(s * LANES, LANES)]
        g3 = vgx2[pl.ds(s * LANES, LANES)]
        g4 = vgy2[pl.ds(s * LANES, LANES)]
        vgarea[pl.ds(s * LANES, LANES)] = (
            jnp.maximum(g3 - g1, zero) * jnp.maximum(g4 - g2, zero))

    iota = lax.iota(jnp.int32, LANES)

    for c in range(CPW):
        a = c * LANES
        bx1 = vx1[pl.ds(a, LANES)]
        by1 = vy1[pl.ds(a, LANES)]
        bx2 = vx2[pl.ds(a, LANES)]
        by2 = vy2[pl.ds(a, LANES)]
        barea = jnp.maximum(bx2 - bx1, zero) * jnp.maximum(by2 - by1, zero)

        def gt_step(j, carry):
            inter_b, union_b, idx_b = carry
            js = jnp.full((LANES,), j, jnp.int32)
            g1 = plsc.load_gather(vgx1, [js])
            g2 = plsc.load_gather(vgy1, [js])
            g3 = plsc.load_gather(vgx2, [js])
            g4 = plsc.load_gather(vgy2, [js])
            ga = plsc.load_gather(vgarea, [js])
            w = jnp.maximum(jnp.minimum(bx2, g3) - jnp.maximum(bx1, g1), zero)
            h = jnp.maximum(jnp.minimum(by2, g4) - jnp.maximum(by1, g2), zero)
            inter = w * h
            union = barea + ga - inter
            upd = inter * union_b > inter_b * union
            inter_b = jnp.where(upd, inter, inter_b)
            union_b = jnp.where(upd, union, union_b)
            idx_b = jnp.where(upd, js, idx_b)
            return inter_b, union_b, idx_b

        init = (jnp.zeros((LANES,), jnp.float32),
                jnp.ones((LANES,), jnp.float32),
                jnp.zeros((LANES,), jnp.int32))
        inter_b, union_b, idx_b = lax.fori_loop(0, M, gt_step, init)

        iou = inter_b / jnp.maximum(union_b, jnp.float32(1e-10))
        pos = iou >= jnp.float32(POS_IOU_THR)
        neg = iou < jnp.float32(NEG_IOU_THR)

        labg = plsc.load_gather(vglab, [idx_b])
        lab = jnp.where(pos, labg,
                        jnp.where(neg, jnp.zeros((LANES,), jnp.int32),
                                  jnp.full((LANES,), -1, jnp.int32)))
        vlab[pl.ds(a, LANES)] = lab

        neg1 = jnp.full((LANES,), -1.0, jnp.float32)
        o1 = jnp.where(pos, plsc.load_gather(vgx1, [idx_b]), neg1)
        o2 = jnp.where(pos, plsc.load_gather(vgy1, [idx_b]), neg1)
        o3 = jnp.where(pos, plsc.load_gather(vgx2, [idx_b]), neg1)
        o4 = jnp.where(pos, plsc.load_gather(vgy2, [idx_b]), neg1)
        fbase = (iota + a) * 4
        plsc.store_scatter(vbox, [fbase], o1)
        plsc.store_scatter(vbox, [fbase + 1], o2)
        plsc.store_scatter(vbox, [fbase + 2], o3)
        plsc.store_scatter(vbox, [fbase + 3], o4)

    pltpu.sync_copy(vlab, olab_h.at[pl.ds(base, APW)])
    pltpu.sync_copy(vbox, obox_h.at[pl.ds(base * 4, APW * 4)])


def kernel(bboxes, gt_bboxes, gt_labels):
    bp = jnp.concatenate(
        [bboxes, jnp.zeros((NPAD - N, 4), jnp.float32)], axis=0)
    bx1, by1, bx2, by2 = (bp[:, i] for i in range(4))
    gx1, gy1, gx2, gy2 = (gt_bboxes[:, i] for i in range(4))
    olab, obox = _assign(bx1, by1, bx2, by2, gx1, gy1, gx2, gy2,
                         gt_labels.astype(jnp.int32))
    return olab[:N], obox[: N * 4].reshape(N, 4)


# speedup vs baseline: 3.6636x; 1.2415x over previous
import functools

import jax
import jax.numpy as jnp
from jax import lax
from jax.experimental import pallas as pl
from jax.experimental.pallas import tpu as pltpu
from jax.experimental.pallas import tpu_sc as plsc

POS_IOU_THR = 0.5
NEG_IOU_THR = 0.4

N = 20000
M = 128
LANES = 16
NWORKERS = 32
APW = 640
APW_LAST = N - 31 * APW
CPW = APW // LANES
CPW_LAST = APW_LAST // LANES

_mesh = plsc.VectorSubcoreMesh(core_axis_name="c", subcore_axis_name="s")


@functools.partial(
    pl.kernel,
    mesh=_mesh,
    compiler_params=pltpu.CompilerParams(needs_layout_passes=False),
    out_type=[
        jax.ShapeDtypeStruct((N,), jnp.int32),
        jax.ShapeDtypeStruct((N, 128), jnp.float32),
    ],
    scratch_types=[
        pltpu.VMEM((APW,), jnp.float32),
        pltpu.VMEM((APW,), jnp.float32),
        pltpu.VMEM((APW,), jnp.float32),
        pltpu.VMEM((APW,), jnp.float32),
        pltpu.VMEM((M,), jnp.float32),
        pltpu.VMEM((M,), jnp.float32),
        pltpu.VMEM((M,), jnp.float32),
        pltpu.VMEM((M,), jnp.float32),
        pltpu.VMEM((M,), jnp.float32),
        pltpu.VMEM((M,), jnp.int32),
        pltpu.VMEM((APW,), jnp.int32),
        pltpu.VMEM((APW, 128), jnp.float32),
    ],
)
def _assign(bx1_h, by1_h, bx2_h, by2_h, gx1_h, gy1_h, gx2_h, gy2_h, glab_h,
            olab_h, obox_h,
            vx1, vy1, vx2, vy2, vgx1, vgy1, vgx2, vgy2, vgarea, vglab,
            vlab, vbox):
    wid = lax.axis_index("s") * 2 + lax.axis_index("c")
    base = wid * APW
    not_last = wid < NWORKERS - 1

    for src, dst in ((bx1_h, vx1), (by1_h, vy1), (bx2_h, vx2), (by2_h, vy2)):
        pltpu.sync_copy(src.at[pl.ds(base, APW_LAST)], dst.at[pl.ds(0, APW_LAST)])
    pltpu.sync_copy(gx1_h, vgx1)
    pltpu.sync_copy(gy1_h, vgy1)
    pltpu.sync_copy(gx2_h, vgx2)
    pltpu.sync_copy(gy2_h, vgy2)
    pltpu.sync_copy(glab_h, vglab)

    @pl.when(not_last)
    def _copy_rest():
        for src, dst in ((bx1_h, vx1), (by1_h, vy1), (bx2_h, vx2), (by2_h, vy2)):
            pltpu.sync_copy(src.at[pl.ds(base + APW_LAST, APW - APW_LAST)],
                            dst.at[pl.ds(APW_LAST, APW - APW_LAST)])

    zero = jnp.float32(0.0)
    for s in range(M // LANES):
        g1 = vgx1[pl.ds(s * LANES, LANES)]
        g2 = vgy1[pl.ds(s * LANES, LANES)]
        g3 = vgx2[pl.ds(s * LANES, LANES)]
        g4 = vgy2[pl.ds(s * LANES, LANES)]
        vgarea[pl.ds(s * LANES, LANES)] = (
            jnp.maximum(g3 - g1, zero) * jnp.maximum(g4 - g2, zero))

    iota = lax.iota(jnp.int32, LANES)

    def do_chunk_pair(p):
        a0 = (2 * p) * LANES
        a1 = (2 * p + 1) * LANES
        bx1a = vx1[pl.ds(a0, LANES)]
        by1a = vy1[pl.ds(a0, LANES)]
        bx2a = vx2[pl.ds(a0, LANES)]
        by2a = vy2[pl.ds(a0, LANES)]
        bx1b = vx1[pl.ds(a1, LANES)]
        by1b = vy1[pl.ds(a1, LANES)]
        bx2b = vx2[pl.ds(a1, LANES)]
        by2b = vy2[pl.ds(a1, LANES)]
        bareaa = (jnp.maximum(bx2a - bx1a, zero)
                  * jnp.maximum(by2a - by1a, zero))
        bareab = (jnp.maximum(bx2b - bx1b, zero)
                  * jnp.maximum(by2b - by1b, zero))

        def gt_step(j, carry):
            ia, sa, xa, ib, sb, xb = carry
            js = jnp.full((LANES,), j, jnp.int32)
            g1 = plsc.load_gather(vgx1, [js])
            g2 = plsc.load_gather(vgy1, [js])
            g3 = plsc.load_gather(vgx2, [js])
            g4 = plsc.load_gather(vgy2, [js])
            ga = plsc.load_gather(vgarea, [js])

            wa = jnp.maximum(jnp.minimum(bx2a, g3) - jnp.maximum(bx1a, g1), zero)
            ha = jnp.maximum(jnp.minimum(by2a, g4) - jnp.maximum(by1a, g2), zero)
            intera = wa * ha
            ssa = bareaa + ga
            upda = intera * sa > ia * ssa
            ia = jnp.where(upda, intera, ia)
            sa = jnp.where(upda, ssa, sa)
            xa = jnp.where(upda, js, xa)

            wb = jnp.maximum(jnp.minimum(bx2b, g3) - jnp.maximum(bx1b, g1), zero)
            hb = jnp.maximum(jnp.minimum(by2b, g4) - jnp.maximum(by1b, g2), zero)
            interb = wb * hb
            ssb = bareab + ga
            updb = interb * sb > ib * ssb
            ib = jnp.where(updb, interb, ib)
            sb = jnp.where(updb, ssb, sb)
            xb = jnp.where(updb, js, xb)
            return ia, sa, xa, ib, sb, xb

        init = (jnp.zeros((LANES,), jnp.float32),
                jnp.ones((LANES,), jnp.float32),
                jnp.zeros((LANES,), jnp.int32)) * 2
        ia, sa, xa, ib, sb, xb = lax.fori_loop(0, M, gt_step, init)

        for a, inter_b, s_b, idx_b in ((a0, ia, sa, xa), (a1, ib, sb, xb)):
            union_b = s_b - inter_b
            iou = inter_b / jnp.maximum(union_b, jnp.float32(1e-10))
            pos = iou >= jnp.float32(POS_IOU_THR)
            neg = iou < jnp.float32(NEG_IOU_THR)

            labg = plsc.load_gather(vglab, [idx_b])
            lab = jnp.where(pos, labg,
                            jnp.where(neg, jnp.zeros((LANES,), jnp.int32),
                                      jnp.full((LANES,), -1, jnp.int32)))
            vlab[pl.ds(a, LANES)] = lab

            neg1 = jnp.full((LANES,), -1.0, jnp.float32)
            rows = iota + a
            for c, src in enumerate((vgx1, vgy1, vgx2, vgy2)):
                oc = jnp.where(pos, plsc.load_gather(src, [idx_b]), neg1)
                plsc.store_scatter(
                    vbox, [rows, jnp.full((LANES,), c, jnp.int32)], oc)

    for p in range(CPW_LAST // 2):
        do_chunk_pair(p)

    @pl.when(not_last)
    def _rest_chunks():
        for p in range(CPW_LAST // 2, CPW // 2):
            do_chunk_pair(p)

    pltpu.sync_copy(vlab.at[pl.ds(0, APW_LAST)],
                    olab_h.at[pl.ds(base, APW_LAST)])
    pltpu.sync_copy(vbox.at[pl.ds(0, APW_LAST)],
                    obox_h.at[pl.ds(base, APW_LAST)])

    @pl.when(not_last)
    def _copy_out_rest():
        pltpu.sync_copy(vlab.at[pl.ds(APW_LAST, APW - APW_LAST)],
                        olab_h.at[pl.ds(base + APW_LAST, APW - APW_LAST)])
        pltpu.sync_copy(vbox.at[pl.ds(APW_LAST, APW - APW_LAST)],
                        obox_h.at[pl.ds(base + APW_LAST, APW - APW_LAST)])


def kernel(bboxes, gt_bboxes, gt_labels):
    bx1, by1, bx2, by2 = (bboxes[:, i] for i in range(4))
    gx1, gy1, gx2, gy2 = (gt_bboxes[:, i] for i in range(4))
    olab, obox = _assign(bx1, by1, bx2, by2, gx1, gy1, gx2, gy2,
                         gt_labels.astype(jnp.int32))
    return olab, obox[:, :4]


# speedup vs baseline: 3.7109x; 1.0129x over previous
import functools

import jax
import jax.numpy as jnp
from jax import lax
from jax.experimental import pallas as pl
from jax.experimental.pallas import tpu as pltpu
from jax.experimental.pallas import tpu_sc as plsc

POS_IOU_THR = 0.5
NEG_IOU_THR = 0.4

N = 20000
M = 128
LANES = 16
NWORKERS = 32
APW = 640
APW_LAST = N - 31 * APW
CPW = APW // LANES
GROUP = 4

_mesh = plsc.VectorSubcoreMesh(core_axis_name="c", subcore_axis_name="s")


@functools.partial(
    pl.kernel,
    mesh=_mesh,
    compiler_params=pltpu.CompilerParams(needs_layout_passes=False),
    out_type=[
        jax.ShapeDtypeStruct((N,), jnp.int32),
        jax.ShapeDtypeStruct((N, 128), jnp.float32),
    ],
    scratch_types=[
        pltpu.VMEM((APW,), jnp.float32),
        pltpu.VMEM((APW,), jnp.float32),
        pltpu.VMEM((APW,), jnp.float32),
        pltpu.VMEM((APW,), jnp.float32),
        pltpu.VMEM((M,), jnp.float32),
        pltpu.VMEM((M,), jnp.float32),
        pltpu.VMEM((M,), jnp.float32),
        pltpu.VMEM((M,), jnp.float32),
        pltpu.VMEM((M,), jnp.float32),
        pltpu.VMEM((M,), jnp.int32),
        pltpu.VMEM((APW,), jnp.int32),
        pltpu.VMEM((APW, 128), jnp.float32),
    ],
)
def _assign(bx1_h, by1_h, bx2_h, by2_h, gx1_h, gy1_h, gx2_h, gy2_h, glab_h,
            olab_h, obox_h,
            vx1, vy1, vx2, vy2, vgx1, vgy1, vgx2, vgy2, vgarea, vglab,
            vlab, vbox):
    wid = lax.axis_index("s") * 2 + lax.axis_index("c")
    base = wid * APW
    not_last = wid < NWORKERS - 1

    for src, dst in ((bx1_h, vx1), (by1_h, vy1), (bx2_h, vx2), (by2_h, vy2)):
        pltpu.sync_copy(src.at[pl.ds(base, APW_LAST)], dst.at[pl.ds(0, APW_LAST)])
    pltpu.sync_copy(gx1_h, vgx1)
    pltpu.sync_copy(gy1_h, vgy1)
    pltpu.sync_copy(gx2_h, vgx2)
    pltpu.sync_copy(gy2_h, vgy2)
    pltpu.sync_copy(glab_h, vglab)

    @pl.when(not_last)
    def _copy_rest():
        for src, dst in ((bx1_h, vx1), (by1_h, vy1), (bx2_h, vx2), (by2_h, vy2)):
            pltpu.sync_copy(src.at[pl.ds(base + APW_LAST, APW - APW_LAST)],
                            dst.at[pl.ds(APW_LAST, APW - APW_LAST)])

    zero = jnp.float32(0.0)
    for s in range(M // LANES):
        g1 = vgx1[pl.ds(s * LANES, LANES)]
        g2 = vgy1[pl.ds(s * LANES, LANES)]
        g3 = vgx2[pl.ds(s * LANES, LANES)]
        g4 = vgy2[pl.ds(s * LANES, LANES)]
        vgarea[pl.ds(s * LANES, LANES)] = (
            jnp.maximum(g3 - g1, zero) * jnp.maximum(g4 - g2, zero))

    iota = lax.iota(jnp.int32, LANES)

    def do_group(g):
        cs = [g * GROUP + k for k in range(GROUP)]
        bx1 = [vx1[pl.ds(c * LANES, LANES)] for c in cs]
        by1 = [vy1[pl.ds(c * LANES, LANES)] for c in cs]
        bx2 = [vx2[pl.ds(c * LANES, LANES)] for c in cs]
        by2 = [vy2[pl.ds(c * LANES, LANES)] for c in cs]
        barea = [jnp.maximum(bx2[k] - bx1[k], zero)
                 * jnp.maximum(by2[k] - by1[k], zero) for k in range(GROUP)]

        def gt_step(t, carry):
            st = list(carry)
            for u in range(2):
                j = 2 * t + u
                js = jnp.full((LANES,), j, jnp.int32)
                g1 = plsc.load_gather(vgx1, [js])
                g2 = plsc.load_gather(vgy1, [js])
                g3 = plsc.load_gather(vgx2, [js])
                g4 = plsc.load_gather(vgy2, [js])
                ga = plsc.load_gather(vgarea, [js])
                for k in range(GROUP):
                    ib, sb, xb = st[3 * k], st[3 * k + 1], st[3 * k + 2]
                    w = jnp.maximum(
                        jnp.minimum(bx2[k], g3) - jnp.maximum(bx1[k], g1), zero)
                    h = jnp.maximum(
                        jnp.minimum(by2[k], g4) - jnp.maximum(by1[k], g2), zero)
                    inter = w * h
                    ss = barea[k] + ga
                    upd = inter * sb > ib * ss
                    st[3 * k] = jnp.where(upd, inter, ib)
                    st[3 * k + 1] = jnp.where(upd, ss, sb)
                    st[3 * k + 2] = jnp.where(upd, js, xb)
            return tuple(st)

        init = (jnp.zeros((LANES,), jnp.float32),
                jnp.ones((LANES,), jnp.float32),
                jnp.zeros((LANES,), jnp.int32)) * GROUP
        st = lax.fori_loop(0, M // 2, gt_step, init)

        for k in range(GROUP):
            a = cs[k] * LANES
            inter_b, s_b, idx_b = st[3 * k], st[3 * k + 1], st[3 * k + 2]
            union_b = s_b - inter_b
            iou = inter_b / jnp.maximum(union_b, jnp.float32(1e-10))
            pos = iou >= jnp.float32(POS_IOU_THR)
            neg = iou < jnp.float32(NEG_IOU_THR)

            labg = plsc.load_gather(vglab, [idx_b])
            lab = jnp.where(pos, labg,
                            jnp.where(neg, jnp.zeros((LANES,), jnp.int32),
                                      jnp.full((LANES,), -1, jnp.int32)))
            vlab[pl.ds(a, LANES)] = lab

            neg1 = jnp.full((LANES,), -1.0, jnp.float32)
            rows = iota + a
            for c, src in enumerate((vgx1, vgy1, vgx2, vgy2)):
                oc = jnp.where(pos, plsc.load_gather(src, [idx_b]), neg1)
                plsc.store_scatter(
                    vbox, [rows, jnp.full((LANES,), c, jnp.int32)], oc)

    for g in range(CPW // GROUP):
        do_group(g)

    pltpu.sync_copy(vlab.at[pl.ds(0, APW_LAST)],
                    olab_h.at[pl.ds(base, APW_LAST)])
    pltpu.sync_copy(vbox.at[pl.ds(0, APW_LAST)],
                    obox_h.at[pl.ds(base, APW_LAST)])

    @pl.when(not_last)
    def _copy_out_rest():
        pltpu.sync_copy(vlab.at[pl.ds(APW_LAST, APW - APW_LAST)],
                        olab_h.at[pl.ds(base + APW_LAST, APW - APW_LAST)])
        pltpu.sync_copy(vbox.at[pl.ds(APW_LAST, APW - APW_LAST)],
                        obox_h.at[pl.ds(base + APW_LAST, APW - APW_LAST)])


def kernel(bboxes, gt_bboxes, gt_labels):
    bx1, by1, bx2, by2 = (bboxes[:, i] for i in range(4))
    gx1, gy1, gx2, gy2 = (gt_bboxes[:, i] for i in range(4))
    olab, obox = _assign(bx1, by1, bx2, by2, gx1, gy1, gx2, gy2,
                         gt_labels.astype(jnp.int32))
    return olab, obox[:, :4]


# speedup vs baseline: 4.0947x; 1.1034x over previous
import functools

import jax
import jax.numpy as jnp
from jax import lax
from jax.experimental import pallas as pl
from jax.experimental.pallas import tpu as pltpu
from jax.experimental.pallas import tpu_sc as plsc

POS_IOU_THR = 0.5
NEG_IOU_THR = 0.4

N = 20000
M = 128
LANES = 16
NWORKERS = 32
APW = 640
APW_LAST = N - 31 * APW
CPW = APW // LANES
GROUP = 4

_mesh = plsc.VectorSubcoreMesh(core_axis_name="c", subcore_axis_name="s")


@functools.partial(
    pl.kernel,
    mesh=_mesh,
    compiler_params=pltpu.CompilerParams(needs_layout_passes=False),
    out_type=[
        jax.ShapeDtypeStruct((N,), jnp.int32),
        jax.ShapeDtypeStruct((N, 128), jnp.float32),
    ],
    scratch_types=[
        pltpu.VMEM((APW,), jnp.float32),
        pltpu.VMEM((APW,), jnp.float32),
        pltpu.VMEM((APW,), jnp.float32),
        pltpu.VMEM((APW,), jnp.float32),
        pltpu.VMEM((M,), jnp.float32),
        pltpu.VMEM((M,), jnp.float32),
        pltpu.VMEM((M,), jnp.float32),
        pltpu.VMEM((M,), jnp.float32),
        pltpu.VMEM((M,), jnp.float32),
        pltpu.VMEM((M,), jnp.int32),
        pltpu.VMEM((APW,), jnp.int32),
        pltpu.VMEM((APW, 128), jnp.float32),
        pltpu.SemaphoreType.DMA,
    ],
)
def _assign(bx1_h, by1_h, bx2_h, by2_h, gx1_h, gy1_h, gx2_h, gy2_h, glab_h,
            olab_h, obox_h,
            vx1, vy1, vx2, vy2, vgx1, vgy1, vgx2, vgy2, vgarea, vglab,
            vlab, vbox, sem):
    wid = lax.axis_index("s") * 2 + lax.axis_index("c")
    base = wid * APW
    not_last = wid < NWORKERS - 1

    copies = [
        pltpu.make_async_copy(src.at[pl.ds(base, APW_LAST)],
                              dst.at[pl.ds(0, APW_LAST)], sem)
        for src, dst in ((bx1_h, vx1), (by1_h, vy1), (bx2_h, vx2), (by2_h, vy2))
    ] + [
        pltpu.make_async_copy(src, dst, sem)
        for src, dst in ((gx1_h, vgx1), (gy1_h, vgy1), (gx2_h, vgx2),
                         (gy2_h, vgy2), (glab_h, vglab))
    ]
    for cp in copies:
        cp.start()

    rest = [
        pltpu.make_async_copy(src.at[pl.ds(base + APW_LAST, APW - APW_LAST)],
                              dst.at[pl.ds(APW_LAST, APW - APW_LAST)], sem)
        for src, dst in ((bx1_h, vx1), (by1_h, vy1), (bx2_h, vx2), (by2_h, vy2))
    ]

    @pl.when(not_last)
    def _copy_rest():
        for cp in rest:
            cp.start()

    for cp in copies:
        cp.wait()

    @pl.when(not_last)
    def _wait_rest():
        for cp in rest:
            cp.wait()

    zero = jnp.float32(0.0)
    for s in range(M // LANES):
        g1 = vgx1[pl.ds(s * LANES, LANES)]
        g2 = vgy1[pl.ds(s * LANES, LANES)]
        g3 = vgx2[pl.ds(s * LANES, LANES)]
        g4 = vgy2[pl.ds(s * LANES, LANES)]
        vgarea[pl.ds(s * LANES, LANES)] = (
            jnp.maximum(g3 - g1, zero) * jnp.maximum(g4 - g2, zero))

    iota = lax.iota(jnp.int32, LANES)

    def do_group(g):
        cs = [g * GROUP + k for k in range(GROUP)]
        bx1 = [vx1[pl.ds(c * LANES, LANES)] for c in cs]
        by1 = [vy1[pl.ds(c * LANES, LANES)] for c in cs]
        bx2 = [vx2[pl.ds(c * LANES, LANES)] for c in cs]
        by2 = [vy2[pl.ds(c * LANES, LANES)] for c in cs]
        barea = [jnp.maximum(bx2[k] - bx1[k], zero)
                 * jnp.maximum(by2[k] - by1[k], zero) for k in range(GROUP)]

        def gt_step(t, carry):
            st = list(carry)
            for u in range(2):
                j = 2 * t + u
                js = jnp.full((LANES,), j, jnp.int32)
                g1 = plsc.load_gather(vgx1, [js])
                g2 = plsc.load_gather(vgy1, [js])
                g3 = plsc.load_gather(vgx2, [js])
                g4 = plsc.load_gather(vgy2, [js])
                ga = plsc.load_gather(vgarea, [js])
                for k in range(GROUP):
                    ib, sb, xb = st[3 * k], st[3 * k + 1], st[3 * k + 2]
                    w = jnp.maximum(
                        jnp.minimum(bx2[k], g3) - jnp.maximum(bx1[k], g1), zero)
                    h = jnp.maximum(
                        jnp.minimum(by2[k], g4) - jnp.maximum(by1[k], g2), zero)
                    inter = w * h
                    ss = barea[k] + ga
                    upd = inter * sb > ib * ss
                    st[3 * k] = jnp.where(upd, inter, ib)
                    st[3 * k + 1] = jnp.where(upd, ss, sb)
                    st[3 * k + 2] = jnp.where(upd, js, xb)
            return tuple(st)

        init = (jnp.zeros((LANES,), jnp.float32),
                jnp.ones((LANES,), jnp.float32),
                jnp.zeros((LANES,), jnp.int32)) * GROUP
        st = lax.fori_loop(0, M // 2, gt_step, init)

        for k in range(GROUP):
            a = cs[k] * LANES
            inter_b, s_b, idx_b = st[3 * k], st[3 * k + 1], st[3 * k + 2]
            union_b = s_b - inter_b
            iou = inter_b / jnp.maximum(union_b, jnp.float32(1e-10))
            pos = iou >= jnp.float32(POS_IOU_THR)
            neg = iou < jnp.float32(NEG_IOU_THR)

            labg = plsc.load_gather(vglab, [idx_b])
            lab = jnp.where(pos, labg,
                            jnp.where(neg, jnp.zeros((LANES,), jnp.int32),
                                      jnp.full((LANES,), -1, jnp.int32)))
            vlab[pl.ds(a, LANES)] = lab

            neg1 = jnp.full((LANES,), -1.0, jnp.float32)
            rows = iota + a
            for c, src in enumerate((vgx1, vgy1, vgx2, vgy2)):
                oc = jnp.where(pos, plsc.load_gather(src, [idx_b]), neg1)
                plsc.store_scatter(
                    vbox, [rows, jnp.full((LANES,), c, jnp.int32)], oc)

    for g in range(CPW // GROUP):
        do_group(g)

    out_first = [
        pltpu.make_async_copy(vlab.at[pl.ds(0, APW_LAST)],
                              olab_h.at[pl.ds(base, APW_LAST)], sem),
        pltpu.make_async_copy(vbox.at[pl.ds(0, APW_LAST)],
                              obox_h.at[pl.ds(base, APW_LAST)], sem),
    ]
    out_rest = [
        pltpu.make_async_copy(vlab.at[pl.ds(APW_LAST, APW - APW_LAST)],
                              olab_h.at[pl.ds(base + APW_LAST, APW - APW_LAST)],
                              sem),
        pltpu.make_async_copy(vbox.at[pl.ds(APW_LAST, APW - APW_LAST)],
                              obox_h.at[pl.ds(base + APW_LAST, APW - APW_LAST)],
                              sem),
    ]
    for cp in out_first:
        cp.start()

    @pl.when(not_last)
    def _copy_out_rest():
        for cp in out_rest:
            cp.start()

    for cp in out_first:
        cp.wait()

    @pl.when(not_last)
    def _wait_out_rest():
        for cp in out_rest:
            cp.wait()


def kernel(bboxes, gt_bboxes, gt_labels):
    bx1, by1, bx2, by2 = (bboxes[:, i] for i in range(4))
    gx1, gy1, gx2, gy2 = (gt_bboxes[:, i] for i in range(4))
    olab, obox = _assign(bx1, by1, bx2, by2, gx1, gy1, gx2, gy2,
                         gt_labels.astype(jnp.int32))
    return olab, obox[:, :4]


# speedup vs baseline: 4.2104x; 1.0283x over previous
import functools

import jax
import jax.numpy as jnp
from jax import lax
from jax.experimental import pallas as pl
from jax.experimental.pallas import tpu as pltpu
from jax.experimental.pallas import tpu_sc as plsc

POS_IOU_THR = 0.5
NEG_IOU_THR = 0.4

N = 20000
M = 128
LANES = 16
NWORKERS = 32
APW = 640
APW_LAST = N - 31 * APW
CPW = APW // LANES
GROUP = 5

_mesh = plsc.VectorSubcoreMesh(core_axis_name="c", subcore_axis_name="s")


@functools.partial(
    pl.kernel,
    mesh=_mesh,
    compiler_params=pltpu.CompilerParams(needs_layout_passes=False),
    out_type=[
        jax.ShapeDtypeStruct((N,), jnp.int32),
        jax.ShapeDtypeStruct((N, 128), jnp.float32),
    ],
    scratch_types=[
        pltpu.VMEM((APW,), jnp.float32),
        pltpu.VMEM((APW,), jnp.float32),
        pltpu.VMEM((APW,), jnp.float32),
        pltpu.VMEM((APW,), jnp.float32),
        pltpu.VMEM((M,), jnp.float32),
        pltpu.VMEM((M,), jnp.float32),
        pltpu.VMEM((M,), jnp.float32),
        pltpu.VMEM((M,), jnp.float32),
        pltpu.VMEM((M,), jnp.float32),
        pltpu.VMEM((M,), jnp.int32),
        pltpu.VMEM((APW,), jnp.int32),
        pltpu.VMEM((APW, 128), jnp.float32),
        pltpu.SemaphoreType.DMA,
    ],
)
def _assign(bx1_h, by1_h, bx2_h, by2_h, gx1_h, gy1_h, gx2_h, gy2_h, glab_h,
            olab_h, obox_h,
            vx1, vy1, vx2, vy2, vgx1, vgy1, vgx2, vgy2, vgarea, vglab,
            vlab, vbox, sem):
    wid = lax.axis_index("s") * 2 + lax.axis_index("c")
    base = wid * APW
    not_last = wid < NWORKERS - 1

    copies = [
        pltpu.make_async_copy(src.at[pl.ds(base, APW_LAST)],
                              dst.at[pl.ds(0, APW_LAST)], sem)
        for src, dst in ((bx1_h, vx1), (by1_h, vy1), (bx2_h, vx2), (by2_h, vy2))
    ] + [
        pltpu.make_async_copy(src, dst, sem)
        for src, dst in ((gx1_h, vgx1), (gy1_h, vgy1), (gx2_h, vgx2),
                         (gy2_h, vgy2), (glab_h, vglab))
    ]
    for cp in copies:
        cp.start()

    rest = [
        pltpu.make_async_copy(src.at[pl.ds(base + APW_LAST, APW - APW_LAST)],
                              dst.at[pl.ds(APW_LAST, APW - APW_LAST)], sem)
        for src, dst in ((bx1_h, vx1), (by1_h, vy1), (bx2_h, vx2), (by2_h, vy2))
    ]

    @pl.when(not_last)
    def _copy_rest():
        for cp in rest:
            cp.start()

    for cp in copies:
        cp.wait()

    @pl.when(not_last)
    def _wait_rest():
        for cp in rest:
            cp.wait()

    zero = jnp.float32(0.0)
    for s in range(M // LANES):
        g1 = vgx1[pl.ds(s * LANES, LANES)]
        g2 = vgy1[pl.ds(s * LANES, LANES)]
        g3 = vgx2[pl.ds(s * LANES, LANES)]
        g4 = vgy2[pl.ds(s * LANES, LANES)]
        vgarea[pl.ds(s * LANES, LANES)] = (
            jnp.maximum(g3 - g1, zero) * jnp.maximum(g4 - g2, zero))

    iota = lax.iota(jnp.int32, LANES)

    def do_group(g):
        cs = [g * GROUP + k for k in range(GROUP)]
        bx1 = [vx1[pl.ds(c * LANES, LANES)] for c in cs]
        by1 = [vy1[pl.ds(c * LANES, LANES)] for c in cs]
        bx2 = [vx2[pl.ds(c * LANES, LANES)] for c in cs]
        by2 = [vy2[pl.ds(c * LANES, LANES)] for c in cs]
        barea = [jnp.maximum(bx2[k] - bx1[k], zero)
                 * jnp.maximum(by2[k] - by1[k], zero) for k in range(GROUP)]

        def gt_step(t, carry):
            st = list(carry)
            for u in range(2):
                j = 2 * t + u
                js = jnp.full((LANES,), j, jnp.int32)
                g1 = plsc.load_gather(vgx1, [js])
                g2 = plsc.load_gather(vgy1, [js])
                g3 = plsc.load_gather(vgx2, [js])
                g4 = plsc.load_gather(vgy2, [js])
                ga = plsc.load_gather(vgarea, [js])
                for k in range(GROUP):
                    ib, sb, xb = st[3 * k], st[3 * k + 1], st[3 * k + 2]
                    w = jnp.maximum(
                        jnp.minimum(bx2[k], g3) - jnp.maximum(bx1[k], g1), zero)
                    h = jnp.minimum(by2[k], g4) - jnp.maximum(by1[k], g2)
                    inter = w * h
                    ss = barea[k] + ga
                    upd = inter * sb > ib * ss
                    st[3 * k] = jnp.where(upd, inter, ib)
                    st[3 * k + 1] = jnp.where(upd, ss, sb)
                    st[3 * k + 2] = jnp.where(upd, js, xb)
            return tuple(st)

        init = (jnp.zeros((LANES,), jnp.float32),
                jnp.ones((LANES,), jnp.float32),
                jnp.zeros((LANES,), jnp.int32)) * GROUP
        st = lax.fori_loop(0, M // 2, gt_step, init)

        for k in range(GROUP):
            a = cs[k] * LANES
            inter_b, s_b, idx_b = st[3 * k], st[3 * k + 1], st[3 * k + 2]
            union_b = s_b - inter_b
            iou = inter_b / jnp.maximum(union_b, jnp.float32(1e-10))
            pos = iou >= jnp.float32(POS_IOU_THR)
            neg = iou < jnp.float32(NEG_IOU_THR)

            labg = plsc.load_gather(vglab, [idx_b])
            lab = jnp.where(pos, labg,
                            jnp.where(neg, jnp.zeros((LANES,), jnp.int32),
                                      jnp.full((LANES,), -1, jnp.int32)))
            vlab[pl.ds(a, LANES)] = lab

            neg1 = jnp.full((LANES,), -1.0, jnp.float32)
            rows = iota + a
            for c, src in enumerate((vgx1, vgy1, vgx2, vgy2)):
                oc = jnp.where(pos, plsc.load_gather(src, [idx_b]), neg1)
                plsc.store_scatter(
                    vbox, [rows, jnp.full((LANES,), c, jnp.int32)], oc)

    for g in range(CPW // GROUP):
        do_group(g)

    out_first = [
        pltpu.make_async_copy(vlab.at[pl.ds(0, APW_LAST)],
                              olab_h.at[pl.ds(base, APW_LAST)], sem),
        pltpu.make_async_copy(vbox.at[pl.ds(0, APW_LAST)],
                              obox_h.at[pl.ds(base, APW_LAST)], sem),
    ]
    out_rest = [
        pltpu.make_async_copy(vlab.at[pl.ds(APW_LAST, APW - APW_LAST)],
                              olab_h.at[pl.ds(base + APW_LAST, APW - APW_LAST)],
                              sem),
        pltpu.make_async_copy(vbox.at[pl.ds(APW_LAST, APW - APW_LAST)],
                              obox_h.at[pl.ds(base + APW_LAST, APW - APW_LAST)],
                              sem),
    ]
    for cp in out_first:
        cp.start()

    @pl.when(not_last)
    def _copy_out_rest():
        for cp in out_rest:
            cp.start()

    for cp in out_first:
        cp.wait()

    @pl.when(not_last)
    def _wait_out_rest():
        for cp in out_rest:
            cp.wait()


def kernel(bboxes, gt_bboxes, gt_labels):
    bx1, by1, bx2, by2 = (bboxes[:, i] for i in range(4))
    gx1, gy1, gx2, gy2 = (gt_bboxes[:, i] for i in range(4))
    olab, obox = _assign(bx1, by1, bx2, by2, gx1, gy1, gx2, gy2,
                         gt_labels.astype(jnp.int32))
    return olab, obox[:, :4]


# speedup vs baseline: 4.4248x; 1.0509x over previous
import functools

import jax
import jax.numpy as jnp
from jax import lax
from jax.experimental import pallas as pl
from jax.experimental.pallas import tpu as pltpu
from jax.experimental.pallas import tpu_sc as plsc

POS_IOU_THR = 0.5
NEG_IOU_THR = 0.4

N = 20000
M = 128
LANES = 16
NWORKERS = 32
APW = 640
APW_LAST = N - 31 * APW
CPW = APW // LANES
GROUP = 5

_mesh = plsc.VectorSubcoreMesh(core_axis_name="c", subcore_axis_name="s")


@functools.partial(
    pl.kernel,
    mesh=_mesh,
    compiler_params=pltpu.CompilerParams(needs_layout_passes=False),
    out_type=[
        jax.ShapeDtypeStruct((N,), jnp.int32),
        jax.ShapeDtypeStruct((N, 128), jnp.float32),
    ],
    scratch_types=[
        pltpu.VMEM((APW,), jnp.float32),
        pltpu.VMEM((APW,), jnp.float32),
        pltpu.VMEM((APW,), jnp.float32),
        pltpu.VMEM((APW,), jnp.float32),
        pltpu.VMEM((M,), jnp.float32),
        pltpu.VMEM((M,), jnp.float32),
        pltpu.VMEM((M,), jnp.float32),
        pltpu.VMEM((M,), jnp.float32),
        pltpu.VMEM((M,), jnp.float32),
        pltpu.VMEM((M,), jnp.int32),
        pltpu.VMEM((APW,), jnp.int32),
        pltpu.VMEM((APW, 128), jnp.float32),
        pltpu.SemaphoreType.DMA,
    ],
)
def _assign(bx1_h, by1_h, bx2_h, by2_h, gx1_h, gy1_h, gx2_h, gy2_h, glab_h,
            olab_h, obox_h,
            vx1, vy1, vx2, vy2, vgx1, vgy1, vgx2, vgy2, vgarea, vglab,
            vlab, vbox, sem):
    wid = lax.axis_index("s") * 2 + lax.axis_index("c")
    base = wid * APW
    not_last = wid < NWORKERS - 1

    copies = [
        pltpu.make_async_copy(src.at[pl.ds(base, APW_LAST)],
                              dst.at[pl.ds(0, APW_LAST)], sem)
        for src, dst in ((bx1_h, vx1), (by1_h, vy1), (bx2_h, vx2), (by2_h, vy2))
    ] + [
        pltpu.make_async_copy(src, dst, sem)
        for src, dst in ((gx1_h, vgx1), (gy1_h, vgy1), (gx2_h, vgx2),
                         (gy2_h, vgy2), (glab_h, vglab))
    ]
    for cp in copies:
        cp.start()

    rest = [
        pltpu.make_async_copy(src.at[pl.ds(base + APW_LAST, APW - APW_LAST)],
                              dst.at[pl.ds(APW_LAST, APW - APW_LAST)], sem)
        for src, dst in ((bx1_h, vx1), (by1_h, vy1), (bx2_h, vx2), (by2_h, vy2))
    ]

    @pl.when(not_last)
    def _copy_rest():
        for cp in rest:
            cp.start()

    for cp in copies:
        cp.wait()

    @pl.when(not_last)
    def _wait_rest():
        for cp in rest:
            cp.wait()

    zero = jnp.float32(0.0)
    for s in range(M // LANES):
        g1 = vgx1[pl.ds(s * LANES, LANES)]
        g2 = vgy1[pl.ds(s * LANES, LANES)]
        g3 = vgx2[pl.ds(s * LANES, LANES)]
        g4 = vgy2[pl.ds(s * LANES, LANES)]
        vgarea[pl.ds(s * LANES, LANES)] = (
            jnp.maximum(g3 - g1, zero) * jnp.maximum(g4 - g2, zero))

    iota = lax.iota(jnp.int32, LANES)

    def do_group(g, _):
        gbase = g * (GROUP * LANES)
        offs = [gbase + k * LANES for k in range(GROUP)]
        bx1 = [vx1[pl.ds(o, LANES)] for o in offs]
        by1 = [vy1[pl.ds(o, LANES)] for o in offs]
        bx2 = [vx2[pl.ds(o, LANES)] for o in offs]
        by2 = [vy2[pl.ds(o, LANES)] for o in offs]
        barea = [jnp.maximum(bx2[k] - bx1[k], zero)
                 * jnp.maximum(by2[k] - by1[k], zero) for k in range(GROUP)]

        def gt_step(t, carry):
            st = list(carry)
            for u in range(2):
                j = 2 * t + u
                js = jnp.full((LANES,), j, jnp.int32)
                g1 = plsc.load_gather(vgx1, [js])
                g2 = plsc.load_gather(vgy1, [js])
                g3 = plsc.load_gather(vgx2, [js])
                g4 = plsc.load_gather(vgy2, [js])
                ga = plsc.load_gather(vgarea, [js])
                for k in range(GROUP):
                    ib, sb, xb = st[3 * k], st[3 * k + 1], st[3 * k + 2]
                    w = jnp.maximum(
                        jnp.minimum(bx2[k], g3) - jnp.maximum(bx1[k], g1), zero)
                    h = jnp.minimum(by2[k], g4) - jnp.maximum(by1[k], g2)
                    inter = w * h
                    ss = barea[k] + ga
                    upd = inter * sb > ib * ss
                    st[3 * k] = jnp.where(upd, inter, ib)
                    st[3 * k + 1] = jnp.where(upd, ss, sb)
                    st[3 * k + 2] = jnp.where(upd, js, xb)
            return tuple(st)

        init = (jnp.zeros((LANES,), jnp.float32),
                jnp.ones((LANES,), jnp.float32),
                jnp.zeros((LANES,), jnp.int32)) * GROUP
        st = lax.fori_loop(0, M // 2, gt_step, init)

        for k in range(GROUP):
            a = offs[k]
            inter_b, s_b, idx_b = st[3 * k], st[3 * k + 1], st[3 * k + 2]
            union_b = s_b - inter_b
            iou = inter_b / jnp.maximum(union_b, jnp.float32(1e-10))
            pos = iou >= jnp.float32(POS_IOU_THR)
            neg = iou < jnp.float32(NEG_IOU_THR)

            labg = plsc.load_gather(vglab, [idx_b])
            lab = jnp.where(pos, labg,
                            jnp.where(neg, jnp.zeros((LANES,), jnp.int32),
                                      jnp.full((LANES,), -1, jnp.int32)))
            vlab[pl.ds(a, LANES)] = lab

            neg1 = jnp.full((LANES,), -1.0, jnp.float32)
            rows = iota + a
            for c, src in enumerate((vgx1, vgy1, vgx2, vgy2)):
                oc = jnp.where(pos, plsc.load_gather(src, [idx_b]), neg1)
                plsc.store_scatter(
                    vbox, [rows, jnp.full((LANES,), c, jnp.int32)], oc)

    lax.fori_loop(0, CPW // GROUP, do_group, None)

    out_first = [
        pltpu.make_async_copy(vlab.at[pl.ds(0, APW_LAST)],
                              olab_h.at[pl.ds(base, APW_LAST)], sem),
        pltpu.make_async_copy(vbox.at[pl.ds(0, APW_LAST)],
                              obox_h.at[pl.ds(base, APW_LAST)], sem),
    ]
    out_rest = [
        pltpu.make_async_copy(vlab.at[pl.ds(APW_LAST, APW - APW_LAST)],
                              olab_h.at[pl.ds(base + APW_LAST, APW - APW_LAST)],
                              sem),
        pltpu.make_async_copy(vbox.at[pl.ds(APW_LAST, APW - APW_LAST)],
                              obox_h.at[pl.ds(base + APW_LAST, APW - APW_LAST)],
                              sem),
    ]
    for cp in out_first:
        cp.start()

    @pl.when(not_last)
    def _copy_out_rest():
        for cp in out_rest:
            cp.start()

    for cp in out_first:
        cp.wait()

    @pl.when(not_last)
    def _wait_out_rest():
        for cp in out_rest:
            cp.wait()


def kernel(bboxes, gt_bboxes, gt_labels):
    bx1, by1, bx2, by2 = (bboxes[:, i] for i in range(4))
    gx1, gy1, gx2, gy2 = (gt_bboxes[:, i] for i in range(4))
    olab, obox = _assign(bx1, by1, bx2, by2, gx1, gy1, gx2, gy2,
                         gt_labels.astype(jnp.int32))
    return olab, obox[:, :4]


# speedup vs baseline: 4.4345x; 1.0022x over previous
import functools

import jax
import jax.numpy as jnp
from jax import lax
from jax.experimental import pallas as pl
from jax.experimental.pallas import tpu as pltpu
from jax.experimental.pallas import tpu_sc as plsc

POS_IOU_THR = 0.5
NEG_IOU_THR = 0.4

N = 20000
M = 128
LANES = 16
NWORKERS = 32
APW = 640
APW_LAST = N - 31 * APW
CPW = APW // LANES
GROUP = 5

_mesh = plsc.VectorSubcoreMesh(core_axis_name="c", subcore_axis_name="s")


@functools.partial(
    pl.kernel,
    mesh=_mesh,
    compiler_params=pltpu.CompilerParams(needs_layout_passes=False),
    out_type=[
        jax.ShapeDtypeStruct((N,), jnp.int32),
        jax.ShapeDtypeStruct((N, 4), jnp.float32),
    ],
    scratch_types=[
        pltpu.VMEM((APW,), jnp.float32),
        pltpu.VMEM((APW,), jnp.float32),
        pltpu.VMEM((APW,), jnp.float32),
        pltpu.VMEM((APW,), jnp.float32),
        pltpu.VMEM((M,), jnp.float32),
        pltpu.VMEM((M,), jnp.float32),
        pltpu.VMEM((M,), jnp.float32),
        pltpu.VMEM((M,), jnp.float32),
        pltpu.VMEM((M,), jnp.float32),
        pltpu.VMEM((M,), jnp.int32),
        pltpu.VMEM((APW,), jnp.int32),
        pltpu.VMEM((APW, 4), jnp.float32),
        pltpu.SemaphoreType.DMA,
    ],
)
def _assign(bx1_h, by1_h, bx2_h, by2_h, gx1_h, gy1_h, gx2_h, gy2_h, glab_h,
            olab_h, obox_h,
            vx1, vy1, vx2, vy2, vgx1, vgy1, vgx2, vgy2, vgarea, vglab,
            vlab, vbox, sem):
    wid = lax.axis_index("s") * 2 + lax.axis_index("c")
    base = wid * APW
    not_last = wid < NWORKERS - 1

    copies = [
        pltpu.make_async_copy(src.at[pl.ds(base, APW_LAST)],
                              dst.at[pl.ds(0, APW_LAST)], sem)
        for src, dst in ((bx1_h, vx1), (by1_h, vy1), (bx2_h, vx2), (by2_h, vy2))
    ] + [
        pltpu.make_async_copy(src, dst, sem)
        for src, dst in ((gx1_h, vgx1), (gy1_h, vgy1), (gx2_h, vgx2),
                         (gy2_h, vgy2), (glab_h, vglab))
    ]
    for cp in copies:
        cp.start()

    rest = [
        pltpu.make_async_copy(src.at[pl.ds(base + APW_LAST, APW - APW_LAST)],
                              dst.at[pl.ds(APW_LAST, APW - APW_LAST)], sem)
        for src, dst in ((bx1_h, vx1), (by1_h, vy1), (bx2_h, vx2), (by2_h, vy2))
    ]

    @pl.when(not_last)
    def _copy_rest():
        for cp in rest:
            cp.start()

    for cp in copies:
        cp.wait()

    @pl.when(not_last)
    def _wait_rest():
        for cp in rest:
            cp.wait()

    zero = jnp.float32(0.0)
    for s in range(M // LANES):
        g1 = vgx1[pl.ds(s * LANES, LANES)]
        g2 = vgy1[pl.ds(s * LANES, LANES)]
        g3 = vgx2[pl.ds(s * LANES, LANES)]
        g4 = vgy2[pl.ds(s * LANES, LANES)]
        vgarea[pl.ds(s * LANES, LANES)] = (
            jnp.maximum(g3 - g1, zero) * jnp.maximum(g4 - g2, zero))

    iota = lax.iota(jnp.int32, LANES)

    def do_group(g, _):
        gbase = g * (GROUP * LANES)
        offs = [gbase + k * LANES for k in range(GROUP)]
        bx1 = [vx1[pl.ds(o, LANES)] for o in offs]
        by1 = [vy1[pl.ds(o, LANES)] for o in offs]
        bx2 = [vx2[pl.ds(o, LANES)] for o in offs]
        by2 = [vy2[pl.ds(o, LANES)] for o in offs]
        barea = [jnp.maximum(bx2[k] - bx1[k], zero)
                 * jnp.maximum(by2[k] - by1[k], zero) for k in range(GROUP)]

        def gt_step(t, carry):
            st = list(carry)
            for u in range(2):
                j = 2 * t + u
                js = jnp.full((LANES,), j, jnp.int32)
                g1 = plsc.load_gather(vgx1, [js])
                g2 = plsc.load_gather(vgy1, [js])
                g3 = plsc.load_gather(vgx2, [js])
                g4 = plsc.load_gather(vgy2, [js])
                ga = plsc.load_gather(vgarea, [js])
                for k in range(GROUP):
                    ib, sb, xb = st[3 * k], st[3 * k + 1], st[3 * k + 2]
                    w = jnp.maximum(
                        jnp.minimum(bx2[k], g3) - jnp.maximum(bx1[k], g1), zero)
                    h = jnp.minimum(by2[k], g4) - jnp.maximum(by1[k], g2)
                    inter = w * h
                    ss = barea[k] + ga
                    upd = inter * sb > ib * ss
                    st[3 * k] = jnp.where(upd, inter, ib)
                    st[3 * k + 1] = jnp.where(upd, ss, sb)
                    st[3 * k + 2] = jnp.where(upd, js, xb)
            return tuple(st)

        init = (jnp.zeros((LANES,), jnp.float32),
                jnp.ones((LANES,), jnp.float32),
                jnp.zeros((LANES,), jnp.int32)) * GROUP
        st = lax.fori_loop(0, M // 2, gt_step, init)

        for k in range(GROUP):
            a = offs[k]
            inter_b, s_b, idx_b = st[3 * k], st[3 * k + 1], st[3 * k + 2]
            union_b = s_b - inter_b
            iou = inter_b / jnp.maximum(union_b, jnp.float32(1e-10))
            pos = iou >= jnp.float32(POS_IOU_THR)
            neg = iou < jnp.float32(NEG_IOU_THR)

            labg = plsc.load_gather(vglab, [idx_b])
            lab = jnp.where(pos, labg,
                            jnp.where(neg, jnp.zeros((LANES,), jnp.int32),
                                      jnp.full((LANES,), -1, jnp.int32)))
            vlab[pl.ds(a, LANES)] = lab

            neg1 = jnp.full((LANES,), -1.0, jnp.float32)
            rows = iota + a
            for c, src in enumerate((vgx1, vgy1, vgx2, vgy2)):
                oc = jnp.where(pos, plsc.load_gather(src, [idx_b]), neg1)
                plsc.store_scatter(
                    vbox, [rows, jnp.full((LANES,), c, jnp.int32)], oc)

    lax.fori_loop(0, CPW // GROUP, do_group, None)

    out_first = [
        pltpu.make_async_copy(vlab.at[pl.ds(0, APW_LAST)],
                              olab_h.at[pl.ds(base, APW_LAST)], sem),
        pltpu.make_async_copy(vbox.at[pl.ds(0, APW_LAST)],
                              obox_h.at[pl.ds(base, APW_LAST)], sem),
    ]
    out_rest = [
        pltpu.make_async_copy(vlab.at[pl.ds(APW_LAST, APW - APW_LAST)],
                              olab_h.at[pl.ds(base + APW_LAST, APW - APW_LAST)],
                              sem),
        pltpu.make_async_copy(vbox.at[pl.ds(APW_LAST, APW - APW_LAST)],
                              obox_h.at[pl.ds(base + APW_LAST, APW - APW_LAST)],
                              sem),
    ]
    for cp in out_first:
        cp.start()

    @pl.when(not_last)
    def _copy_out_rest():
        for cp in out_rest:
            cp.start()

    for cp in out_first:
        cp.wait()

    @pl.when(not_last)
    def _wait_out_rest():
        for cp in out_rest:
            cp.wait()


def kernel(bboxes, gt_bboxes, gt_labels):
    bx1, by1, bx2, by2 = (bboxes[:, i] for i in range(4))
    gx1, gy1, gx2, gy2 = (gt_bboxes[:, i] for i in range(4))
    olab, obox = _assign(bx1, by1, bx2, by2, gx1, gy1, gx2, gy2,
                         gt_labels.astype(jnp.int32))
    return olab, obox


# speedup vs baseline: 4.6628x; 1.0515x over previous
import functools

import jax
import jax.numpy as jnp
from jax import lax
from jax.experimental import pallas as pl
from jax.experimental.pallas import tpu as pltpu
from jax.experimental.pallas import tpu_sc as plsc

POS_IOU_THR = 0.5
NEG_IOU_THR = 0.4

N = 20000
M = 128
LANES = 16
NWORKERS = 32
APW = 640
APW_LAST = N - 31 * APW
CPW = APW // LANES
GROUP = 5

_mesh = plsc.VectorSubcoreMesh(core_axis_name="c", subcore_axis_name="s")


@functools.partial(
    pl.kernel,
    mesh=_mesh,
    compiler_params=pltpu.CompilerParams(needs_layout_passes=False),
    out_type=[
        jax.ShapeDtypeStruct((N,), jnp.int32),
        jax.ShapeDtypeStruct((N, 4), jnp.float32),
    ],
    scratch_types=[
        pltpu.VMEM((APW,), jnp.float32),
        pltpu.VMEM((APW,), jnp.float32),
        pltpu.VMEM((APW,), jnp.float32),
        pltpu.VMEM((APW,), jnp.float32),
        pltpu.VMEM((M,), jnp.float32),
        pltpu.VMEM((M,), jnp.float32),
        pltpu.VMEM((M,), jnp.float32),
        pltpu.VMEM((M,), jnp.float32),
        pltpu.VMEM((M,), jnp.float32),
        pltpu.VMEM((M,), jnp.int32),
        pltpu.VMEM((APW,), jnp.int32),
        pltpu.VMEM((APW, 4), jnp.float32),
        pltpu.SemaphoreType.DMA,
    ],
)
def _assign(bx1_h, by1_h, bx2_h, by2_h, gx1_h, gy1_h, gx2_h, gy2_h, glab_h,
            olab_h, obox_h,
            vx1, vy1, vx2, vy2, vgx1, vgy1, vgx2, vgy2, vgarea, vglab,
            vlab, vbox, sem):
    wid = lax.axis_index("s") * 2 + lax.axis_index("c")
    base = wid * APW
    not_last = wid < NWORKERS - 1

    copies = [
        pltpu.make_async_copy(src.at[pl.ds(base, APW_LAST)],
                              dst.at[pl.ds(0, APW_LAST)], sem)
        for src, dst in ((bx1_h, vx1), (by1_h, vy1), (bx2_h, vx2), (by2_h, vy2))
    ] + [
        pltpu.make_async_copy(src, dst, sem)
        for src, dst in ((gx1_h, vgx1), (gy1_h, vgy1), (gx2_h, vgx2),
                         (gy2_h, vgy2), (glab_h, vglab))
    ]
    for cp in copies:
        cp.start()

    rest = [
        pltpu.make_async_copy(src.at[pl.ds(base + APW_LAST, APW - APW_LAST)],
                              dst.at[pl.ds(APW_LAST, APW - APW_LAST)], sem)
        for src, dst in ((bx1_h, vx1), (by1_h, vy1), (bx2_h, vx2), (by2_h, vy2))
    ]

    @pl.when(not_last)
    def _copy_rest():
        for cp in rest:
            cp.start()

    for cp in copies:
        cp.wait()

    @pl.when(not_last)
    def _wait_rest():
        for cp in rest:
            cp.wait()

    zero = jnp.float32(0.0)
    for s in range(M // LANES):
        g1 = vgx1[pl.ds(s * LANES, LANES)]
        g2 = vgy1[pl.ds(s * LANES, LANES)]
        g3 = vgx2[pl.ds(s * LANES, LANES)]
        g4 = vgy2[pl.ds(s * LANES, LANES)]
        vgarea[pl.ds(s * LANES, LANES)] = (
            jnp.maximum(g3 - g1, zero) * jnp.maximum(g4 - g2, zero))

    iota = lax.iota(jnp.int32, LANES)

    def do_group(g, _):
        gbase = g * (GROUP * LANES)
        offs = [gbase + k * LANES for k in range(GROUP)]
        bx1 = [vx1[pl.ds(o, LANES)] for o in offs]
        by1 = [vy1[pl.ds(o, LANES)] for o in offs]
        bx2 = [vx2[pl.ds(o, LANES)] for o in offs]
        by2 = [vy2[pl.ds(o, LANES)] for o in offs]
        barea = [jnp.maximum(bx2[k] - bx1[k], zero)
                 * jnp.maximum(by2[k] - by1[k], zero) for k in range(GROUP)]

        def gt_step(t, carry):
            st = list(carry)
            for u in range(2):
                j = 2 * t + u
                js = jnp.full((LANES,), j, jnp.int32)
                g1 = plsc.load_gather(vgx1, [js])
                g2 = plsc.load_gather(vgy1, [js])
                g3 = plsc.load_gather(vgx2, [js])
                g4 = plsc.load_gather(vgy2, [js])
                ga = plsc.load_gather(vgarea, [js])
                for k in range(GROUP):
                    ib, sb, xb = st[3 * k], st[3 * k + 1], st[3 * k + 2]
                    w = jnp.maximum(
                        jnp.minimum(bx2[k], g3) - jnp.maximum(bx1[k], g1), zero)
                    h = jnp.minimum(by2[k], g4) - jnp.maximum(by1[k], g2)
                    inter = w * h
                    ss = barea[k] + ga
                    upd = inter * sb > ib * ss
                    st[3 * k] = jnp.where(upd, inter, ib)
                    st[3 * k + 1] = jnp.where(upd, ss, sb)
                    st[3 * k + 2] = jnp.where(upd, js, xb)
            return tuple(st)

        init = (jnp.zeros((LANES,), jnp.float32),
                jnp.ones((LANES,), jnp.float32),
                jnp.zeros((LANES,), jnp.int32)) * GROUP
        st = lax.fori_loop(0, M // 2, gt_step, init)

        for k in range(GROUP):
            a = offs[k]
            inter_b, s_b, idx_b = st[3 * k], st[3 * k + 1], st[3 * k + 2]
            union_b = s_b - inter_b
            iou = inter_b / jnp.maximum(union_b, jnp.float32(1e-10))
            pos = iou >= jnp.float32(POS_IOU_THR)
            neg = iou < jnp.float32(NEG_IOU_THR)

            labg = plsc.load_gather(vglab, [idx_b])
            lab = jnp.where(pos, labg,
                            jnp.where(neg, jnp.zeros((LANES,), jnp.int32),
                                      jnp.full((LANES,), -1, jnp.int32)))
            vlab[pl.ds(a, LANES)] = lab

            neg1 = jnp.full((LANES,), -1.0, jnp.float32)
            rows = iota + a
            for c, src in enumerate((vgx1, vgy1, vgx2, vgy2)):
                oc = jnp.where(pos, plsc.load_gather(src, [idx_b]), neg1)
                plsc.store_scatter(
                    vbox, [rows, jnp.full((LANES,), c, jnp.int32)], oc)

        apg = GROUP * LANES

        @pl.when(jnp.logical_or(not_last, g < APW_LAST // apg))
        def _out_dma():
            pltpu.make_async_copy(vlab.at[pl.ds(gbase, apg)],
                                  olab_h.at[pl.ds(base + gbase, apg)],
                                  sem).start()
            pltpu.make_async_copy(vbox.at[pl.ds(gbase, apg)],
                                  obox_h.at[pl.ds(base + gbase, apg)],
                                  sem).start()

    lax.fori_loop(0, CPW // GROUP, do_group, None)

    for gs in range(CPW // GROUP):
        apg = GROUP * LANES

        @pl.when(jnp.logical_or(not_last, gs < APW_LAST // apg))
        def _out_wait():
            pltpu.make_async_copy(vlab.at[pl.ds(gs * apg, apg)],
                                  olab_h.at[pl.ds(base + gs * apg, apg)],
                                  sem).wait()
            pltpu.make_async_copy(vbox.at[pl.ds(gs * apg, apg)],
                                  obox_h.at[pl.ds(base + gs * apg, apg)],
                                  sem).wait()


def kernel(bboxes, gt_bboxes, gt_labels):
    bx1, by1, bx2, by2 = (bboxes[:, i] for i in range(4))
    gx1, gy1, gx2, gy2 = (gt_bboxes[:, i] for i in range(4))
    olab, obox = _assign(bx1, by1, bx2, by2, gx1, gy1, gx2, gy2,
                         gt_labels.astype(jnp.int32))
    return olab, obox
